# Initial kernel scaffold; baseline (speedup 1.0000x reference)
import jax
import jax.numpy as jnp
from jax.experimental import pallas as pl


def kernel(x, edge_index, W1, b1, W2, b2, Wo, bo):
    raise NotImplementedError("write your pallas kernel here")


# speedup vs baseline: 12.5027x; 12.5027x over previous
import functools

import jax
import jax.numpy as jnp
from jax import lax
from jax.experimental import pallas as pl
from jax.experimental.pallas import tpu as pltpu
from jax.experimental.pallas import tpu_sc as plsc

N = 10000
E = 320000
D = 128

NC = 2
NS = 16
NW = NC * NS
EPW = E // NW
K = 80
NCHUNK = EPW // K
NP = 10240
DSEG = NP // NS
RPS = NP // NS
ZROWS = 32

_MESH = plsc.VectorSubcoreMesh(core_axis_name="c", subcore_axis_name="s")


@functools.partial(
    pl.kernel,
    out_type=jax.ShapeDtypeStruct((NC, NP), jnp.float32),
    mesh=_MESH,
    scratch_types=[
        pltpu.VMEM((K,), jnp.int32),
        pltpu.VMEM((K,), jnp.float32),
        pltpu.VMEM((DSEG,), jnp.float32),
        pltpu.VMEM_SHARED((NP,), jnp.float32),
    ],
)
def _deg_kernel(dst_hbm, out_hbm, didx, ones_v, zbuf, dacc):
    cid = lax.axis_index("c")
    sid = lax.axis_index("s")
    wid = sid * NC + cid
    zero16 = jnp.zeros((16,), jnp.float32)
    ones16 = jnp.ones((16,), jnp.float32)

    def zfill(i, _):
        zbuf[pl.ds(i * 16, 16)] = zero16
        return 0

    lax.fori_loop(0, DSEG // 16, zfill, 0)

    def ofill(i, _):
        ones_v[pl.ds(i * 16, 16)] = ones16
        return 0

    lax.fori_loop(0, K // 16, ofill, 0)

    pltpu.sync_copy(zbuf, dacc.at[pl.ds(sid * DSEG, DSEG)])
    plsc.subcore_barrier()

    base = wid * EPW

    def body(j, _):
        pltpu.sync_copy(dst_hbm.at[pl.ds(base + j * K, K)], didx)
        pltpu.sync_copy(ones_v, dacc.at[didx], add=True)
        return 0

    lax.fori_loop(0, NCHUNK, body, 0)

    plsc.subcore_barrier()
    pltpu.sync_copy(dacc.at[pl.ds(sid * DSEG, DSEG)],
                    out_hbm.at[cid, pl.ds(sid * DSEG, DSEG)])


@functools.partial(
    pl.kernel,
    out_type=jax.ShapeDtypeStruct((NC, NP, D), jnp.float32),
    mesh=_MESH,
    scratch_types=[
        pltpu.VMEM((K,), jnp.int32),
        pltpu.VMEM((K,), jnp.int32),
        pltpu.VMEM((K, D), jnp.float32),
        pltpu.VMEM((ZROWS, D), jnp.float32),
        pltpu.VMEM_SHARED((NP, D), jnp.float32),
        pltpu.SemaphoreType.DMA,
    ],
)
def _agg_kernel(h_hbm, src_hbm, dst_hbm, out_hbm, sidx, didx, rows, zbuf, acc, sem):
    cid = lax.axis_index("c")
    sid = lax.axis_index("s")
    wid = sid * NC + cid
    zero16 = jnp.zeros((16,), jnp.float32)

    def zfill(i, _):
        zbuf[i // (D // 16), pl.ds((i % (D // 16)) * 16, 16)] = zero16
        return 0

    lax.fori_loop(0, ZROWS * (D // 16), zfill, 0)

    def zcopy(j, _):
        pltpu.sync_copy(zbuf, acc.at[pl.ds(sid * RPS + j * ZROWS, ZROWS)])
        return 0

    lax.fori_loop(0, RPS // ZROWS, zcopy, 0)
    plsc.subcore_barrier()

    base = wid * EPW

    def body(j, _):
        pltpu.sync_copy(src_hbm.at[pl.ds(base + j * K, K)], sidx)
        pltpu.sync_copy(dst_hbm.at[pl.ds(base + j * K, K)], didx)
        pltpu.async_copy(h_hbm.at[sidx], rows, sem).wait()
        pltpu.sync_copy(rows, acc.at[didx], add=True)
        return 0

    lax.fori_loop(0, NCHUNK, body, 0)

    plsc.subcore_barrier()
    pltpu.sync_copy(acc.at[pl.ds(sid * RPS, RPS)],
                    out_hbm.at[cid, pl.ds(sid * RPS, RPS)])


BN = 400


def _scale_mm(x, w, degp_t):
    def body(x_ref, w_ref, dp_ref, h_ref, dinv_ref):
        deg = dp_ref[:, 0] + dp_ref[:, 1] + 1.0
        dinv = lax.rsqrt(deg)
        h = jnp.dot(x_ref[...], w_ref[...], preferred_element_type=jnp.float32)
        h_ref[...] = h * dinv[:, None]
        dinv_ref[...] = dinv[:, None]

    return pl.pallas_call(
        body,
        grid=(N // BN,),
        in_specs=[
            pl.BlockSpec((BN, D), lambda i: (i, 0)),
            pl.BlockSpec((D, D), lambda i: (0, 0)),
            pl.BlockSpec((BN, NC), lambda i: (i, 0)),
        ],
        out_specs=[
            pl.BlockSpec((BN, D), lambda i: (i, 0)),
            pl.BlockSpec((BN, 1), lambda i: (i, 0)),
        ],
        out_shape=[
            jax.ShapeDtypeStruct((N, D), jnp.float32),
            jax.ShapeDtypeStruct((N, 1), jnp.float32),
        ],
    )(x, w, degp_t)


def _combine_mm(a0, a1, hp, dinv, b, w):
    def body(a0_ref, a1_ref, hp_ref, dinv_ref, b_ref, w_ref, out_ref):
        dv = dinv_ref[...]
        u = dv * (a0_ref[...] + a1_ref[...] + hp_ref[...]) + b_ref[...]
        u = jnp.maximum(u, 0.0)
        out_ref[...] = jnp.dot(u, w_ref[...],
                               preferred_element_type=jnp.float32) * dv

    return pl.pallas_call(
        body,
        grid=(N // BN,),
        in_specs=[
            pl.BlockSpec((BN, D), lambda i: (i, 0)),
            pl.BlockSpec((BN, D), lambda i: (i, 0)),
            pl.BlockSpec((BN, D), lambda i: (i, 0)),
            pl.BlockSpec((BN, 1), lambda i: (i, 0)),
            pl.BlockSpec((1, D), lambda i: (0, 0)),
            pl.BlockSpec((D, D), lambda i: (0, 0)),
        ],
        out_specs=pl.BlockSpec((BN, D), lambda i: (i, 0)),
        out_shape=jax.ShapeDtypeStruct((N, D), jnp.float32),
    )(a0, a1, hp, dinv, b, w)


def _final_mm(a0, a1, hp, dinv, b, wo, bo):
    def body(a0_ref, a1_ref, hp_ref, dinv_ref, b_ref, wo_ref, bo_ref, out_ref):
        dv = dinv_ref[...]
        v = dv * (a0_ref[...] + a1_ref[...] + hp_ref[...]) + b_ref[...]
        v = jnp.maximum(v, 0.0)
        out_ref[...] = jnp.dot(v, wo_ref[...],
                               preferred_element_type=jnp.float32) + bo_ref[...]

    return pl.pallas_call(
        body,
        grid=(N // BN,),
        in_specs=[
            pl.BlockSpec((BN, D), lambda i: (i, 0)),
            pl.BlockSpec((BN, D), lambda i: (i, 0)),
            pl.BlockSpec((BN, D), lambda i: (i, 0)),
            pl.BlockSpec((BN, 1), lambda i: (i, 0)),
            pl.BlockSpec((1, D), lambda i: (0, 0)),
            pl.BlockSpec((D, 1), lambda i: (0, 0)),
            pl.BlockSpec((1, 1), lambda i: (0, 0)),
        ],
        out_specs=pl.BlockSpec((BN, 1), lambda i: (i, 0)),
        out_shape=jax.ShapeDtypeStruct((N, 1), jnp.float32),
    )(a0, a1, hp, dinv, b, wo, bo)


def kernel(x, edge_index, W1, b1, W2, b2, Wo, bo):
    src = edge_index[0].astype(jnp.int32)
    dst = edge_index[1].astype(jnp.int32)

    degp = _deg_kernel(dst)
    degp_t = degp[:, :N].T

    h1, dinv = _scale_mm(x, W1, degp_t)
    acc1 = _agg_kernel(h1, src, dst)
    h2 = _combine_mm(acc1[0, :N], acc1[1, :N], h1, dinv, b1.reshape(1, D), W2)
    acc2 = _agg_kernel(h2, src, dst)
    out = _final_mm(acc2[0, :N], acc2[1, :N], h2, dinv, b2.reshape(1, D), Wo,
                    bo.reshape(1, 1))
    return out.reshape(-1)


# speedup vs baseline: 28.0923x; 2.2469x over previous
import functools

import jax
import jax.numpy as jnp
from jax import lax
from jax.experimental import pallas as pl
from jax.experimental.pallas import tpu as pltpu
from jax.experimental.pallas import tpu_sc as plsc

N = 10000
E = 320000
D = 128

NC = 2
NS = 16
NW = NC * NS
EPW = E // NW
K = 80
NCHUNK = EPW // K
NP = 10240
DSEG = NP // NS
RPS = NP // NS
ZROWS = 32

_MESH = plsc.VectorSubcoreMesh(core_axis_name="c", subcore_axis_name="s")


DEG_RING = 8


@functools.partial(
    pl.kernel,
    out_type=jax.ShapeDtypeStruct((NC, NP), jnp.float32),
    mesh=_MESH,
    scratch_types=[
        pltpu.VMEM((NCHUNK, K), jnp.int32),
        pltpu.VMEM((K,), jnp.float32),
        pltpu.VMEM((DSEG,), jnp.float32),
        pltpu.VMEM_SHARED((NP,), jnp.float32),
        pltpu.SemaphoreType.DMA,
        pltpu.SemaphoreType.DMA,
    ],
)
def _deg_kernel(dst_hbm, out_hbm, didx, ones_v, zbuf, dacc, isem, ssem):
    cid = lax.axis_index("c")
    sid = lax.axis_index("s")
    wid = sid * NC + cid
    zero16 = jnp.zeros((16,), jnp.float32)
    ones16 = jnp.ones((16,), jnp.float32)

    cidx = pltpu.async_copy(dst_hbm.at[wid], didx, isem)

    def zfill(i, _):
        zbuf[pl.ds(i * 16, 16)] = zero16
        return 0

    lax.fori_loop(0, DSEG // 16, zfill, 0)

    def ofill(i, _):
        ones_v[pl.ds(i * 16, 16)] = ones16
        return 0

    lax.fori_loop(0, K // 16, ofill, 0)

    pltpu.sync_copy(zbuf, dacc.at[pl.ds(sid * DSEG, DSEG)])
    cidx.wait()
    plsc.subcore_barrier()

    def body(j, _):
        pltpu.async_copy(ones_v, dacc.at[didx.at[j]], ssem, add=True)

        @pl.when(j >= DEG_RING)
        def _():
            pltpu.make_async_copy(ones_v, dacc.at[didx.at[0]], ssem).wait()

        return 0

    lax.fori_loop(0, NCHUNK, body, 0)

    def drain(j, _):
        pltpu.make_async_copy(ones_v, dacc.at[didx.at[0]], ssem).wait()
        return 0

    lax.fori_loop(0, DEG_RING, drain, 0)
    plsc.subcore_barrier()
    pltpu.sync_copy(dacc.at[pl.ds(sid * DSEG, DSEG)],
                    out_hbm.at[cid, pl.ds(sid * DSEG, DSEG)])


@functools.partial(
    pl.kernel,
    out_type=jax.ShapeDtypeStruct((NC, NP, D), jnp.float32),
    mesh=_MESH,
    scratch_types=[
        pltpu.VMEM((EPW,), jnp.int32),
        pltpu.VMEM((NCHUNK, K), jnp.int32),
        pltpu.VMEM((K, D), jnp.float32),
        pltpu.VMEM((K, D), jnp.float32),
        pltpu.VMEM_SHARED((NP, D), jnp.float32),
        pltpu.SemaphoreType.DMA,
        pltpu.SemaphoreType.DMA,
    ],
)
def _agg_kernel(h_hbm, src_hbm, dst_hbm, out_hbm,
                sidx, didx, buf0, buf1, acc, sem0, sem1):
    cid = lax.axis_index("c")
    sid = lax.axis_index("s")
    wid = sid * NC + cid
    base = wid * EPW
    zero16 = jnp.zeros((16,), jnp.float32)

    cs = pltpu.async_copy(src_hbm.at[pl.ds(base, EPW)], sidx, sem0)
    cd = pltpu.async_copy(dst_hbm.at[wid], didx, sem1)

    def zfill(i, _):
        buf0[i // (D // 16), pl.ds((i % (D // 16)) * 16, 16)] = zero16
        return 0

    lax.fori_loop(0, K * (D // 16), zfill, 0)

    def zcopy(j, _):
        pltpu.sync_copy(buf0, acc.at[pl.ds(sid * RPS + j * K, K)])
        return 0

    lax.fori_loop(0, RPS // K, zcopy, 0)
    cs.wait()
    cd.wait()
    plsc.subcore_barrier()

    def gather(j, buf, sem):
        return pltpu.async_copy(h_hbm.at[sidx.at[pl.ds(j * K, K)]], buf, sem)

    def gwait(buf, sem):
        pltpu.make_async_copy(h_hbm.at[pl.ds(0, K)], buf, sem).wait()

    gather(0, buf0, sem0)

    def pair(t, _):
        j0 = 2 * t
        gather(j0 + 1, buf1, sem1)
        gwait(buf0, sem0)
        pltpu.sync_copy(buf0, acc.at[didx.at[j0]], add=True)
        gather(j0 + 2, buf0, sem0)
        gwait(buf1, sem1)
        pltpu.sync_copy(buf1, acc.at[didx.at[j0 + 1]], add=True)
        return 0

    lax.fori_loop(0, NCHUNK // 2, pair, 0)
    gwait(buf0, sem0)
    pltpu.sync_copy(buf0, acc.at[didx.at[NCHUNK - 1]], add=True)

    plsc.subcore_barrier()
    pltpu.sync_copy(acc.at[pl.ds(sid * RPS, RPS)],
                    out_hbm.at[cid, pl.ds(sid * RPS, RPS)])


BN = 400


def _scale_mm(x, w, degp_t):
    def body(x_ref, w_ref, dp_ref, h_ref, dinv_ref):
        deg = dp_ref[:, 0] + dp_ref[:, 1] + 1.0
        dinv = lax.rsqrt(deg)
        h = jnp.dot(x_ref[...], w_ref[...], preferred_element_type=jnp.float32)
        h_ref[...] = h * dinv[:, None]
        dinv_ref[...] = dinv[:, None]

    return pl.pallas_call(
        body,
        grid=(N // BN,),
        in_specs=[
            pl.BlockSpec((BN, D), lambda i: (i, 0)),
            pl.BlockSpec((D, D), lambda i: (0, 0)),
            pl.BlockSpec((BN, NC), lambda i: (i, 0)),
        ],
        out_specs=[
            pl.BlockSpec((BN, D), lambda i: (i, 0)),
            pl.BlockSpec((BN, 1), lambda i: (i, 0)),
        ],
        out_shape=[
            jax.ShapeDtypeStruct((N, D), jnp.float32),
            jax.ShapeDtypeStruct((N, 1), jnp.float32),
        ],
    )(x, w, degp_t)


def _combine_mm(a0, a1, hp, dinv, b, w):
    def body(a0_ref, a1_ref, hp_ref, dinv_ref, b_ref, w_ref, out_ref):
        dv = dinv_ref[...]
        u = dv * (a0_ref[...] + a1_ref[...] + hp_ref[...]) + b_ref[...]
        u = jnp.maximum(u, 0.0)
        out_ref[...] = jnp.dot(u, w_ref[...],
                               preferred_element_type=jnp.float32) * dv

    return pl.pallas_call(
        body,
        grid=(N // BN,),
        in_specs=[
            pl.BlockSpec((BN, D), lambda i: (i, 0)),
            pl.BlockSpec((BN, D), lambda i: (i, 0)),
            pl.BlockSpec((BN, D), lambda i: (i, 0)),
            pl.BlockSpec((BN, 1), lambda i: (i, 0)),
            pl.BlockSpec((1, D), lambda i: (0, 0)),
            pl.BlockSpec((D, D), lambda i: (0, 0)),
        ],
        out_specs=pl.BlockSpec((BN, D), lambda i: (i, 0)),
        out_shape=jax.ShapeDtypeStruct((N, D), jnp.float32),
    )(a0, a1, hp, dinv, b, w)


def _final_mm(a0, a1, hp, dinv, b, wo, bo):
    def body(a0_ref, a1_ref, hp_ref, dinv_ref, b_ref, wo_ref, bo_ref, out_ref):
        dv = dinv_ref[...]
        v = dv * (a0_ref[...] + a1_ref[...] + hp_ref[...]) + b_ref[...]
        v = jnp.maximum(v, 0.0)
        out_ref[...] = jnp.dot(v, wo_ref[...],
                               preferred_element_type=jnp.float32) + bo_ref[...]

    return pl.pallas_call(
        body,
        grid=(N // BN,),
        in_specs=[
            pl.BlockSpec((BN, D), lambda i: (i, 0)),
            pl.BlockSpec((BN, D), lambda i: (i, 0)),
            pl.BlockSpec((BN, D), lambda i: (i, 0)),
            pl.BlockSpec((BN, 1), lambda i: (i, 0)),
            pl.BlockSpec((1, D), lambda i: (0, 0)),
            pl.BlockSpec((D, 1), lambda i: (0, 0)),
            pl.BlockSpec((1, 1), lambda i: (0, 0)),
        ],
        out_specs=pl.BlockSpec((BN, 1), lambda i: (i, 0)),
        out_shape=jax.ShapeDtypeStruct((N, 1), jnp.float32),
    )(a0, a1, hp, dinv, b, wo, bo)


def kernel(x, edge_index, W1, b1, W2, b2, Wo, bo):
    src = edge_index[0].astype(jnp.int32)
    dst3 = edge_index[1].astype(jnp.int32).reshape(NW, NCHUNK, K)

    degp = _deg_kernel(dst3)
    degp_t = degp[:, :N].T

    h1, dinv = _scale_mm(x, W1, degp_t)
    acc1 = _agg_kernel(h1, src, dst3)
    h2 = _combine_mm(acc1[0, :N], acc1[1, :N], h1, dinv, b1.reshape(1, D), W2)
    acc2 = _agg_kernel(h2, src, dst3)
    out = _final_mm(acc2[0, :N], acc2[1, :N], h2, dinv, b2.reshape(1, D), Wo,
                    bo.reshape(1, 1))
    return out.reshape(-1)


# speedup vs baseline: 28.8946x; 1.0286x over previous
import functools

import jax
import jax.numpy as jnp
from jax import lax
from jax.experimental import pallas as pl
from jax.experimental.pallas import tpu as pltpu
from jax.experimental.pallas import tpu_sc as plsc

N = 10000
E = 320000
D = 128

NC = 2
NS = 16
NW = NC * NS
EPW = E // NW
K = 80
NCHUNK = EPW // K
NP = 10240
DSEG = NP // NS
RPS = NP // NS
ZROWS = 32

_MESH = plsc.VectorSubcoreMesh(core_axis_name="c", subcore_axis_name="s")


DEG_RING = 8
RCH = 25
RND = NCHUNK // RCH


@functools.partial(
    pl.kernel,
    out_type=jax.ShapeDtypeStruct((NC, NP), jnp.float32),
    mesh=_MESH,
    scratch_types=[
        pltpu.VMEM((2, RCH, K), jnp.int32),
        pltpu.VMEM((K,), jnp.float32),
        pltpu.VMEM((DSEG,), jnp.float32),
        pltpu.VMEM_SHARED((NP,), jnp.float32),
        pltpu.SemaphoreType.DMA,
        pltpu.SemaphoreType.DMA,
    ],
)
def _deg_kernel(ei_hbm, out_hbm, didx2, ones_v, zbuf, dacc, isem, ssem):
    cid = lax.axis_index("c")
    sid = lax.axis_index("s")
    wid = sid * NC + cid
    zero16 = jnp.zeros((16,), jnp.float32)
    ones16 = jnp.ones((16,), jnp.float32)

    pltpu.async_copy(ei_hbm.at[1, wid, 0], didx2.at[0], isem)

    def zfill(i, _):
        zbuf[pl.ds(i * 16, 16)] = zero16
        return 0

    lax.fori_loop(0, DSEG // 16, zfill, 0)

    def ofill(i, _):
        ones_v[pl.ds(i * 16, 16)] = ones16
        return 0

    lax.fori_loop(0, K // 16, ofill, 0)

    pltpu.sync_copy(zbuf, dacc.at[pl.ds(sid * DSEG, DSEG)])
    plsc.subcore_barrier()

    def rnd(r, _):
        slot = r % 2
        pltpu.make_async_copy(ei_hbm.at[1, wid, 0],
                              didx2.at[slot], isem).wait()

        @pl.when(r + 1 < RND)
        def _():
            pltpu.async_copy(ei_hbm.at[1, wid, jnp.minimum(r + 1, RND - 1)],
                             didx2.at[1 - slot], isem)

        def sc(i, _):
            pltpu.async_copy(ones_v, dacc.at[didx2.at[slot, i]], ssem,
                             add=True)

            @pl.when(i >= DEG_RING)
            def _():
                pltpu.make_async_copy(ones_v, dacc.at[didx2.at[0, 0]],
                                      ssem).wait()

            return 0

        lax.fori_loop(0, RCH, sc, 0)

        def drain(i, _):
            pltpu.make_async_copy(ones_v, dacc.at[didx2.at[0, 0]],
                                  ssem).wait()
            return 0

        lax.fori_loop(0, DEG_RING, drain, 0)
        return 0

    lax.fori_loop(0, RND, rnd, 0)
    plsc.subcore_barrier()
    pltpu.sync_copy(dacc.at[pl.ds(sid * DSEG, DSEG)],
                    out_hbm.at[cid, pl.ds(sid * DSEG, DSEG)])


@functools.partial(
    pl.kernel,
    out_type=jax.ShapeDtypeStruct((NC, NP, D), jnp.float32),
    mesh=_MESH,
    scratch_types=[
        pltpu.VMEM((EPW,), jnp.int32),
        pltpu.VMEM((NCHUNK, K), jnp.int32),
        pltpu.VMEM((K, D), jnp.float32),
        pltpu.VMEM((K, D), jnp.float32),
        pltpu.VMEM_SHARED((NP, D), jnp.float32),
        pltpu.SemaphoreType.DMA,
        pltpu.SemaphoreType.DMA,
    ],
)
def _agg_kernel(h_hbm, src_hbm, ei_hbm, out_hbm,
                sidx, didx, buf0, buf1, acc, sem0, sem1):
    cid = lax.axis_index("c")
    sid = lax.axis_index("s")
    wid = sid * NC + cid
    base = wid * EPW
    zero16 = jnp.zeros((16,), jnp.float32)

    cs = pltpu.async_copy(src_hbm.at[pl.ds(base, EPW)], sidx, sem0)
    cd = pltpu.async_copy(ei_hbm.at[1, wid], didx, sem1)

    def zfill(i, _):
        buf0[i // (D // 16), pl.ds((i % (D // 16)) * 16, 16)] = zero16
        return 0

    lax.fori_loop(0, K * (D // 16), zfill, 0)

    def zcopy(j, _):
        pltpu.sync_copy(buf0, acc.at[pl.ds(sid * RPS + j * K, K)])
        return 0

    lax.fori_loop(0, RPS // K, zcopy, 0)
    cs.wait()
    cd.wait()
    plsc.subcore_barrier()

    def gather(j, buf, sem):
        return pltpu.async_copy(h_hbm.at[sidx.at[pl.ds(j * K, K)]], buf, sem)

    def gwait(buf, sem):
        pltpu.make_async_copy(h_hbm.at[pl.ds(0, K)], buf, sem).wait()

    gather(0, buf0, sem0)

    def pair(t, _):
        j0 = 2 * t
        gather(j0 + 1, buf1, sem1)
        gwait(buf0, sem0)
        pltpu.sync_copy(buf0, acc.at[didx.at[j0]], add=True)
        gather(j0 + 2, buf0, sem0)
        gwait(buf1, sem1)
        pltpu.sync_copy(buf1, acc.at[didx.at[j0 + 1]], add=True)
        return 0

    lax.fori_loop(0, NCHUNK // 2, pair, 0)
    gwait(buf0, sem0)
    pltpu.sync_copy(buf0, acc.at[didx.at[NCHUNK - 1]], add=True)

    plsc.subcore_barrier()
    pltpu.sync_copy(acc.at[pl.ds(sid * RPS, RPS)],
                    out_hbm.at[cid, pl.ds(sid * RPS, RPS)])


BN = 400


def _scale_mm(x, w, degp):
    def body(x_ref, w_ref, dp_ref, h_ref, dinv_ref):
        deg = dp_ref[:, 0] + dp_ref[:, 1] + 1.0
        dinv = lax.rsqrt(deg)
        h = jnp.dot(x_ref[...], w_ref[...], preferred_element_type=jnp.float32)
        h_ref[...] = h * dinv[:, None]
        dinv_ref[...] = dinv[:, None]

    return pl.pallas_call(
        body,
        grid=(N // BN,),
        in_specs=[
            pl.BlockSpec((BN, D), lambda i: (i, 0)),
            pl.BlockSpec((D, D), lambda i: (0, 0)),
            pl.BlockSpec((BN, NC), lambda i: (i, 0)),
        ],
        out_specs=[
            pl.BlockSpec((BN, D), lambda i: (i, 0)),
            pl.BlockSpec((BN, 1), lambda i: (i, 0)),
        ],
        out_shape=[
            jax.ShapeDtypeStruct((N, D), jnp.float32),
            jax.ShapeDtypeStruct((N, 1), jnp.float32),
        ],
    )(x, w, degp)


def _combine_mm(acc, hp, dinv, b, w):
    def body(a_ref, hp_ref, dinv_ref, b_ref, w_ref, out_ref):
        dv = dinv_ref[...]
        u = dv * (a_ref[0] + a_ref[1] + hp_ref[...]) + b_ref[...]
        u = jnp.maximum(u, 0.0)
        out_ref[...] = jnp.dot(u, w_ref[...],
                               preferred_element_type=jnp.float32) * dv

    return pl.pallas_call(
        body,
        grid=(N // BN,),
        in_specs=[
            pl.BlockSpec((NC, BN, D), lambda i: (0, i, 0)),
            pl.BlockSpec((BN, D), lambda i: (i, 0)),
            pl.BlockSpec((BN, 1), lambda i: (i, 0)),
            pl.BlockSpec((1, D), lambda i: (0, 0)),
            pl.BlockSpec((D, D), lambda i: (0, 0)),
        ],
        out_specs=pl.BlockSpec((BN, D), lambda i: (i, 0)),
        out_shape=jax.ShapeDtypeStruct((N, D), jnp.float32),
    )(acc, hp, dinv, b, w)


def _final_mm(acc, hp, dinv, b, wo, bo):
    def body(a_ref, hp_ref, dinv_ref, b_ref, wo_ref, bo_ref, out_ref):
        dv = dinv_ref[...]
        v = dv * (a_ref[0] + a_ref[1] + hp_ref[...]) + b_ref[...]
        v = jnp.maximum(v, 0.0)
        out_ref[...] = jnp.dot(v, wo_ref[...],
                               preferred_element_type=jnp.float32) + bo_ref[...]

    return pl.pallas_call(
        body,
        grid=(N // BN,),
        in_specs=[
            pl.BlockSpec((NC, BN, D), lambda i: (0, i, 0)),
            pl.BlockSpec((BN, D), lambda i: (i, 0)),
            pl.BlockSpec((BN, 1), lambda i: (i, 0)),
            pl.BlockSpec((1, D), lambda i: (0, 0)),
            pl.BlockSpec((D, 1), lambda i: (0, 0)),
            pl.BlockSpec((1, 1), lambda i: (0, 0)),
        ],
        out_specs=pl.BlockSpec((BN, 1), lambda i: (i, 0)),
        out_shape=jax.ShapeDtypeStruct((N, 1), jnp.float32),
    )(acc, hp, dinv, b, wo, bo)


def kernel(x, edge_index, W1, b1, W2, b2, Wo, bo):
    ei = edge_index.astype(jnp.int32).reshape(2, NW, NCHUNK, K)
    ei5 = ei.reshape(2, NW, RND, RCH, K)

    degp = _deg_kernel(ei5)
    h1, dinv = _scale_mm(x, W1, degp.T)
    src = edge_index[0].astype(jnp.int32)
    acc1 = _agg_kernel(h1, src, ei)
    h2 = _combine_mm(acc1, h1, dinv, b1.reshape(1, D), W2)
    acc2 = _agg_kernel(h2, src, ei)
    out = _final_mm(acc2, h2, dinv, b2.reshape(1, D), Wo, bo.reshape(1, 1))
    return out.reshape(-1)


# speedup vs baseline: 32.9286x; 1.1396x over previous
import functools

import jax
import jax.numpy as jnp
from jax import lax
from jax.experimental import pallas as pl
from jax.experimental.pallas import tpu as pltpu
from jax.experimental.pallas import tpu_sc as plsc

N = 10000
E = 320000
D = 128

NC = 2
NS = 16
NW = NC * NS
EPW = E // NW
K = 80
NCHUNK = EPW // K
NP = 10240
DSEG = NP // NS
RPS = NP // NS
ZROWS = 32

_MESH = plsc.VectorSubcoreMesh(core_axis_name="c", subcore_axis_name="s")


DEG_RING = 8
RCH = 25
RND = NCHUNK // RCH


@functools.partial(
    pl.kernel,
    out_type=jax.ShapeDtypeStruct((NC, NP), jnp.float32),
    mesh=_MESH,
    scratch_types=[
        pltpu.VMEM((2, RCH, K), jnp.int32),
        pltpu.VMEM((K,), jnp.float32),
        pltpu.VMEM((DSEG,), jnp.float32),
        pltpu.VMEM_SHARED((NP,), jnp.float32),
        pltpu.SemaphoreType.DMA,
        pltpu.SemaphoreType.DMA,
    ],
)
def _deg_kernel(ei_hbm, out_hbm, didx2, ones_v, zbuf, dacc, isem, ssem):
    cid = lax.axis_index("c")
    sid = lax.axis_index("s")
    wid = sid * NC + cid
    zero16 = jnp.zeros((16,), jnp.float32)
    ones16 = jnp.ones((16,), jnp.float32)

    pltpu.async_copy(ei_hbm.at[1, wid, 0], didx2.at[0], isem)

    def zfill(i, _):
        zbuf[pl.ds(i * 16, 16)] = zero16
        return 0

    lax.fori_loop(0, DSEG // 16, zfill, 0)

    def ofill(i, _):
        ones_v[pl.ds(i * 16, 16)] = ones16
        return 0

    lax.fori_loop(0, K // 16, ofill, 0)

    pltpu.sync_copy(zbuf, dacc.at[pl.ds(sid * DSEG, DSEG)])
    plsc.subcore_barrier()

    def rnd(r, _):
        slot = r % 2
        pltpu.make_async_copy(ei_hbm.at[1, wid, 0],
                              didx2.at[slot], isem).wait()

        @pl.when(r + 1 < RND)
        def _():
            pltpu.async_copy(ei_hbm.at[1, wid, jnp.minimum(r + 1, RND - 1)],
                             didx2.at[1 - slot], isem)

        def sc(i, _):
            pltpu.async_copy(ones_v, dacc.at[didx2.at[slot, i]], ssem,
                             add=True)

            @pl.when(i >= DEG_RING)
            def _():
                pltpu.make_async_copy(ones_v, dacc.at[didx2.at[0, 0]],
                                      ssem).wait()

            return 0

        lax.fori_loop(0, RCH, sc, 0)

        def drain(i, _):
            pltpu.make_async_copy(ones_v, dacc.at[didx2.at[0, 0]],
                                  ssem).wait()
            return 0

        lax.fori_loop(0, DEG_RING, drain, 0)
        return 0

    lax.fori_loop(0, RND, rnd, 0)
    plsc.subcore_barrier()
    pltpu.sync_copy(dacc.at[pl.ds(sid * DSEG, DSEG)],
                    out_hbm.at[cid, pl.ds(sid * DSEG, DSEG)])


@functools.partial(
    pl.kernel,
    out_type=jax.ShapeDtypeStruct((NC, NP, D), jnp.float32),
    mesh=_MESH,
    scratch_types=[
        pltpu.VMEM((EPW,), jnp.int32),
        pltpu.VMEM((3, K), jnp.int32),
        pltpu.VMEM((K, D), jnp.float32),
        pltpu.VMEM((K, D), jnp.float32),
        pltpu.VMEM((K, D), jnp.float32),
        pltpu.VMEM_SHARED((NP, D), jnp.float32),
        pltpu.SemaphoreType.DMA,
        pltpu.SemaphoreType.DMA,
        pltpu.SemaphoreType.DMA,
        pltpu.SemaphoreType.DMA,
        pltpu.SemaphoreType.DMA,
        pltpu.SemaphoreType.DMA,
    ],
)
def _agg_kernel(h_hbm, src_hbm, ei_hbm, out_hbm,
                sidx, dring, buf0, buf1, buf2, acc,
                gsem0, gsem1, gsem2, isem0, isem1, isem2):
    cid = lax.axis_index("c")
    sid = lax.axis_index("s")
    wid = sid * NC + cid
    base = wid * EPW
    zero16 = jnp.zeros((16,), jnp.float32)
    isems = (isem0, isem1, isem2)

    cs = pltpu.async_copy(src_hbm.at[pl.ds(base, EPW)], sidx, gsem0)
    for s in range(3):
        pltpu.async_copy(ei_hbm.at[1, wid, s], dring.at[s], isems[s])

    def zfill(i, _):
        buf0[i // (D // 16), pl.ds((i % (D // 16)) * 16, 16)] = zero16
        return 0

    lax.fori_loop(0, K * (D // 16), zfill, 0)

    def zcopy(j, _):
        pltpu.sync_copy(buf0, acc.at[pl.ds(sid * RPS + j * K, K)])
        return 0

    lax.fori_loop(0, RPS // K, zcopy, 0)
    cs.wait()
    plsc.subcore_barrier()

    def gather(j, buf, sem):
        jj = jnp.where(j < NCHUNK, j, 0)
        return pltpu.async_copy(h_hbm.at[sidx.at[pl.ds(jj * K, K)]], buf, sem)

    def gwait(buf, sem):
        pltpu.make_async_copy(h_hbm.at[pl.ds(0, K)], buf, sem).wait()

    gather(0, buf0, gsem0)
    gather(1, buf1, gsem1)

    def step(j, slot, buf, sem, nbuf, nsem):
        @pl.when(j + 2 <= NCHUNK)
        def _():
            gather(j + 2, nbuf, nsem)

        gwait(buf, sem)

        @pl.when(j < NCHUNK)
        def _():
            pltpu.make_async_copy(ei_hbm.at[1, wid, 0], dring.at[slot],
                                  isems[slot]).wait()
            pltpu.sync_copy(buf, acc.at[dring.at[slot]], add=True)

        @pl.when(j + 3 < NCHUNK)
        def _():
            pltpu.async_copy(ei_hbm.at[1, wid, j + 3], dring.at[slot],
                             isems[slot])

    def trip(t, _):
        j0 = 3 * t
        step(j0, 0, buf0, gsem0, buf2, gsem2)
        step(j0 + 1, 1, buf1, gsem1, buf0, gsem0)
        step(j0 + 2, 2, buf2, gsem2, buf1, gsem1)
        return 0

    lax.fori_loop(0, (NCHUNK + 2) // 3, trip, 0)

    plsc.subcore_barrier()
    pltpu.sync_copy(acc.at[pl.ds(sid * RPS, RPS)],
                    out_hbm.at[cid, pl.ds(sid * RPS, RPS)])


BN = 400


def _scale_mm(x, w, degp):
    def body(x_ref, w_ref, dp_ref, h_ref, dinv_ref):
        deg = dp_ref[:, 0] + dp_ref[:, 1] + 1.0
        dinv = lax.rsqrt(deg)
        h = jnp.dot(x_ref[...], w_ref[...], preferred_element_type=jnp.float32)
        h_ref[...] = h * dinv[:, None]
        dinv_ref[...] = dinv[:, None]

    return pl.pallas_call(
        body,
        grid=(N // BN,),
        in_specs=[
            pl.BlockSpec((BN, D), lambda i: (i, 0)),
            pl.BlockSpec((D, D), lambda i: (0, 0)),
            pl.BlockSpec((BN, NC), lambda i: (i, 0)),
        ],
        out_specs=[
            pl.BlockSpec((BN, D), lambda i: (i, 0)),
            pl.BlockSpec((BN, 1), lambda i: (i, 0)),
        ],
        out_shape=[
            jax.ShapeDtypeStruct((N, D), jnp.float32),
            jax.ShapeDtypeStruct((N, 1), jnp.float32),
        ],
    )(x, w, degp)


def _combine_mm(acc, hp, dinv, b, w):
    def body(a_ref, hp_ref, dinv_ref, b_ref, w_ref, out_ref):
        dv = dinv_ref[...]
        u = dv * (a_ref[0] + a_ref[1] + hp_ref[...]) + b_ref[...]
        u = jnp.maximum(u, 0.0)
        out_ref[...] = jnp.dot(u, w_ref[...],
                               preferred_element_type=jnp.float32) * dv

    return pl.pallas_call(
        body,
        grid=(N // BN,),
        in_specs=[
            pl.BlockSpec((NC, BN, D), lambda i: (0, i, 0)),
            pl.BlockSpec((BN, D), lambda i: (i, 0)),
            pl.BlockSpec((BN, 1), lambda i: (i, 0)),
            pl.BlockSpec((1, D), lambda i: (0, 0)),
            pl.BlockSpec((D, D), lambda i: (0, 0)),
        ],
        out_specs=pl.BlockSpec((BN, D), lambda i: (i, 0)),
        out_shape=jax.ShapeDtypeStruct((N, D), jnp.float32),
    )(acc, hp, dinv, b, w)


def _final_mm(acc, hp, dinv, b, wo, bo):
    def body(a_ref, hp_ref, dinv_ref, b_ref, wo_ref, bo_ref, out_ref):
        dv = dinv_ref[...]
        v = dv * (a_ref[0] + a_ref[1] + hp_ref[...]) + b_ref[...]
        v = jnp.maximum(v, 0.0)
        out_ref[...] = jnp.dot(v, wo_ref[...],
                               preferred_element_type=jnp.float32) + bo_ref[...]

    return pl.pallas_call(
        body,
        grid=(N // BN,),
        in_specs=[
            pl.BlockSpec((NC, BN, D), lambda i: (0, i, 0)),
            pl.BlockSpec((BN, D), lambda i: (i, 0)),
            pl.BlockSpec((BN, 1), lambda i: (i, 0)),
            pl.BlockSpec((1, D), lambda i: (0, 0)),
            pl.BlockSpec((D, 1), lambda i: (0, 0)),
            pl.BlockSpec((1, 1), lambda i: (0, 0)),
        ],
        out_specs=pl.BlockSpec((BN, 1), lambda i: (i, 0)),
        out_shape=jax.ShapeDtypeStruct((N, 1), jnp.float32),
    )(acc, hp, dinv, b, wo, bo)


def kernel(x, edge_index, W1, b1, W2, b2, Wo, bo):
    ei = edge_index.astype(jnp.int32).reshape(2, NW, NCHUNK, K)
    ei5 = ei.reshape(2, NW, RND, RCH, K)

    degp = _deg_kernel(ei5)
    h1, dinv = _scale_mm(x, W1, degp.T)
    src = edge_index[0].astype(jnp.int32)
    acc1 = _agg_kernel(h1, src, ei)
    h2 = _combine_mm(acc1, h1, dinv, b1.reshape(1, D), W2)
    acc2 = _agg_kernel(h2, src, ei)
    out = _final_mm(acc2, h2, dinv, b2.reshape(1, D), Wo, bo.reshape(1, 1))
    return out.reshape(-1)


# speedup vs baseline: 32.9589x; 1.0009x over previous
import functools

import jax
import jax.numpy as jnp
from jax import lax
from jax.experimental import pallas as pl
from jax.experimental.pallas import tpu as pltpu
from jax.experimental.pallas import tpu_sc as plsc

N = 10000
E = 320000
D = 128

NC = 2
NS = 16
NW = NC * NS
EPW = E // NW
K = 80
NCHUNK = EPW // K
NP = 10240
DSEG = NP // NS
RPS = NP // NS
ZROWS = 32

_MESH = plsc.VectorSubcoreMesh(core_axis_name="c", subcore_axis_name="s")


DEG_RING = 8
RCH = 25
RND = NCHUNK // RCH


@functools.partial(
    pl.kernel,
    out_type=jax.ShapeDtypeStruct((NC, NP), jnp.float32),
    mesh=_MESH,
    scratch_types=[
        pltpu.VMEM((2, RCH, K), jnp.int32),
        pltpu.VMEM((K,), jnp.float32),
        pltpu.VMEM((DSEG,), jnp.float32),
        pltpu.VMEM_SHARED((NP,), jnp.float32),
        pltpu.SemaphoreType.DMA,
        pltpu.SemaphoreType.DMA,
    ],
)
def _deg_kernel(ei_hbm, out_hbm, didx2, ones_v, zbuf, dacc, isem, ssem):
    cid = lax.axis_index("c")
    sid = lax.axis_index("s")
    wid = sid * NC + cid
    zero16 = jnp.zeros((16,), jnp.float32)
    ones16 = jnp.ones((16,), jnp.float32)

    pltpu.async_copy(ei_hbm.at[1, wid, 0], didx2.at[0], isem)

    def zfill(i, _):
        zbuf[pl.ds(i * 16, 16)] = zero16
        return 0

    lax.fori_loop(0, DSEG // 16, zfill, 0)

    def ofill(i, _):
        ones_v[pl.ds(i * 16, 16)] = ones16
        return 0

    lax.fori_loop(0, K // 16, ofill, 0)

    pltpu.sync_copy(zbuf, dacc.at[pl.ds(sid * DSEG, DSEG)])
    plsc.subcore_barrier()

    def rnd(r, _):
        slot = r % 2
        pltpu.make_async_copy(ei_hbm.at[1, wid, 0],
                              didx2.at[slot], isem).wait()

        @pl.when(r + 1 < RND)
        def _():
            pltpu.async_copy(ei_hbm.at[1, wid, jnp.minimum(r + 1, RND - 1)],
                             didx2.at[1 - slot], isem)

        def sc(i, _):
            pltpu.async_copy(ones_v, dacc.at[didx2.at[slot, i]], ssem,
                             add=True)

            @pl.when(i >= DEG_RING)
            def _():
                pltpu.make_async_copy(ones_v, dacc.at[didx2.at[0, 0]],
                                      ssem).wait()

            return 0

        lax.fori_loop(0, RCH, sc, 0)

        def drain(i, _):
            pltpu.make_async_copy(ones_v, dacc.at[didx2.at[0, 0]],
                                  ssem).wait()
            return 0

        lax.fori_loop(0, DEG_RING, drain, 0)
        return 0

    lax.fori_loop(0, RND, rnd, 0)
    plsc.subcore_barrier()
    pltpu.sync_copy(dacc.at[pl.ds(sid * DSEG, DSEG)],
                    out_hbm.at[cid, pl.ds(sid * DSEG, DSEG)])


@functools.partial(
    pl.kernel,
    out_type=jax.ShapeDtypeStruct((NC, NP, D), jnp.float32),
    mesh=_MESH,
    scratch_types=[
        pltpu.VMEM((EPW,), jnp.int32),
        pltpu.VMEM((6, K), jnp.int32),
        pltpu.VMEM((K, D), jnp.float32),
        pltpu.VMEM((K, D), jnp.float32),
        pltpu.VMEM((K, D), jnp.float32),
        pltpu.VMEM_SHARED((NP, D), jnp.float32),
        pltpu.SemaphoreType.DMA,
        pltpu.SemaphoreType.DMA,
        pltpu.SemaphoreType.DMA,
        pltpu.SemaphoreType.DMA,
        pltpu.SemaphoreType.DMA,
        pltpu.SemaphoreType.DMA,
        pltpu.SemaphoreType.DMA,
        pltpu.SemaphoreType.DMA,
        pltpu.SemaphoreType.DMA,
        pltpu.SemaphoreType.DMA,
        pltpu.SemaphoreType.DMA,
        pltpu.SemaphoreType.DMA,
    ],
)
def _agg_kernel(h_hbm, src_hbm, ei_hbm, out_hbm,
                sidx, dring, buf0, buf1, buf2, acc,
                gsem0, gsem1, gsem2, ssem0, ssem1, ssem2,
                isem0, isem1, isem2, isem3, isem4, isem5):
    cid = lax.axis_index("c")
    sid = lax.axis_index("s")
    wid = sid * NC + cid
    base = wid * EPW
    zero16 = jnp.zeros((16,), jnp.float32)
    isems = (isem0, isem1, isem2, isem3, isem4, isem5)
    ssems = (ssem0, ssem1, ssem2)
    bufs = (buf0, buf1, buf2)

    cs = pltpu.async_copy(src_hbm.at[pl.ds(base, EPW)], sidx, gsem0)
    for s in range(3):
        pltpu.async_copy(ei_hbm.at[1, wid, s], dring.at[s], isems[s])

    def zfill(i, _):
        buf0[i // (D // 16), pl.ds((i % (D // 16)) * 16, 16)] = zero16
        return 0

    lax.fori_loop(0, K * (D // 16), zfill, 0)

    def zcopy(j, _):
        pltpu.sync_copy(buf0, acc.at[pl.ds(sid * RPS + j * K, K)])
        return 0

    lax.fori_loop(0, RPS // K, zcopy, 0)
    cs.wait()
    plsc.subcore_barrier()

    def gather(j, buf, sem):
        jj = jnp.where(j < NCHUNK, j, 0)
        return pltpu.async_copy(h_hbm.at[sidx.at[pl.ds(jj * K, K)]], buf, sem)

    def gwait(buf, sem):
        pltpu.make_async_copy(h_hbm.at[pl.ds(0, K)], buf, sem).wait()

    gather(0, buf0, gsem0)
    gather(1, buf1, gsem1)

    def step(j, slot):
        b = slot % 3
        nb = (slot + 2) % 3

        @pl.when(j + 2 <= NCHUNK)
        def _():
            @pl.when(j >= 1)
            def _():
                pltpu.make_async_copy(bufs[nb], acc.at[dring.at[nb]],
                                      ssems[nb]).wait()

            gather(j + 2, bufs[nb], (gsem0, gsem1, gsem2)[nb])

        gwait(bufs[b], (gsem0, gsem1, gsem2)[b])

        @pl.when(j < NCHUNK)
        def _():
            pltpu.make_async_copy(ei_hbm.at[1, wid, 0], dring.at[slot],
                                  isems[slot]).wait()
            pltpu.async_copy(bufs[b], acc.at[dring.at[slot]], ssems[b],
                             add=True)

        @pl.when(j + 3 < NCHUNK)
        def _():
            pltpu.async_copy(ei_hbm.at[1, wid, j + 3],
                             dring.at[(slot + 3) % 6], isems[(slot + 3) % 6])

    def hexa(t, _):
        j0 = 6 * t
        for s in range(6):
            step(j0 + s, s)
        return 0

    lax.fori_loop(0, (NCHUNK + 1) // 6, hexa, 0)
    pltpu.make_async_copy(buf0, acc.at[dring.at[0]], ssem0).wait()
    pltpu.make_async_copy(buf1, acc.at[dring.at[1]], ssem1).wait()

    plsc.subcore_barrier()
    pltpu.sync_copy(acc.at[pl.ds(sid * RPS, RPS)],
                    out_hbm.at[cid, pl.ds(sid * RPS, RPS)])


BN = 400


def _scale_mm(x, w, degp):
    def body(x_ref, w_ref, dp_ref, h_ref, dinv_ref):
        deg = dp_ref[:, 0] + dp_ref[:, 1] + 1.0
        dinv = lax.rsqrt(deg)
        h = jnp.dot(x_ref[...], w_ref[...], preferred_element_type=jnp.float32)
        h_ref[...] = h * dinv[:, None]
        dinv_ref[...] = dinv[:, None]

    return pl.pallas_call(
        body,
        grid=(N // BN,),
        in_specs=[
            pl.BlockSpec((BN, D), lambda i: (i, 0)),
            pl.BlockSpec((D, D), lambda i: (0, 0)),
            pl.BlockSpec((BN, NC), lambda i: (i, 0)),
        ],
        out_specs=[
            pl.BlockSpec((BN, D), lambda i: (i, 0)),
            pl.BlockSpec((BN, 1), lambda i: (i, 0)),
        ],
        out_shape=[
            jax.ShapeDtypeStruct((N, D), jnp.float32),
            jax.ShapeDtypeStruct((N, 1), jnp.float32),
        ],
    )(x, w, degp)


def _combine_mm(acc, hp, dinv, b, w):
    def body(a_ref, hp_ref, dinv_ref, b_ref, w_ref, out_ref):
        dv = dinv_ref[...]
        u = dv * (a_ref[0] + a_ref[1] + hp_ref[...]) + b_ref[...]
        u = jnp.maximum(u, 0.0)
        out_ref[...] = jnp.dot(u, w_ref[...],
                               preferred_element_type=jnp.float32) * dv

    return pl.pallas_call(
        body,
        grid=(N // BN,),
        in_specs=[
            pl.BlockSpec((NC, BN, D), lambda i: (0, i, 0)),
            pl.BlockSpec((BN, D), lambda i: (i, 0)),
            pl.BlockSpec((BN, 1), lambda i: (i, 0)),
            pl.BlockSpec((1, D), lambda i: (0, 0)),
            pl.BlockSpec((D, D), lambda i: (0, 0)),
        ],
        out_specs=pl.BlockSpec((BN, D), lambda i: (i, 0)),
        out_shape=jax.ShapeDtypeStruct((N, D), jnp.float32),
    )(acc, hp, dinv, b, w)


def _final_mm(acc, hp, dinv, b, wo, bo):
    def body(a_ref, hp_ref, dinv_ref, b_ref, wo_ref, bo_ref, out_ref):
        dv = dinv_ref[...]
        v = dv * (a_ref[0] + a_ref[1] + hp_ref[...]) + b_ref[...]
        v = jnp.maximum(v, 0.0)
        out_ref[...] = jnp.dot(v, wo_ref[...],
                               preferred_element_type=jnp.float32) + bo_ref[...]

    return pl.pallas_call(
        body,
        grid=(N // BN,),
        in_specs=[
            pl.BlockSpec((NC, BN, D), lambda i: (0, i, 0)),
            pl.BlockSpec((BN, D), lambda i: (i, 0)),
            pl.BlockSpec((BN, 1), lambda i: (i, 0)),
            pl.BlockSpec((1, D), lambda i: (0, 0)),
            pl.BlockSpec((D, 1), lambda i: (0, 0)),
            pl.BlockSpec((1, 1), lambda i: (0, 0)),
        ],
        out_specs=pl.BlockSpec((BN, 1), lambda i: (i, 0)),
        out_shape=jax.ShapeDtypeStruct((N, 1), jnp.float32),
    )(acc, hp, dinv, b, wo, bo)


def kernel(x, edge_index, W1, b1, W2, b2, Wo, bo):
    ei = edge_index.astype(jnp.int32).reshape(2, NW, NCHUNK, K)
    ei5 = ei.reshape(2, NW, RND, RCH, K)

    degp = _deg_kernel(ei5)
    h1, dinv = _scale_mm(x, W1, degp.T)
    src = edge_index[0].astype(jnp.int32)
    acc1 = _agg_kernel(h1, src, ei)
    h2 = _combine_mm(acc1, h1, dinv, b1.reshape(1, D), W2)
    acc2 = _agg_kernel(h2, src, ei)
    out = _final_mm(acc2, h2, dinv, b2.reshape(1, D), Wo, bo.reshape(1, 1))
    return out.reshape(-1)


# speedup vs baseline: 38.6375x; 1.1723x over previous
import functools

import jax
import jax.numpy as jnp
from jax import lax
from jax.experimental import pallas as pl
from jax.experimental.pallas import tpu as pltpu
from jax.experimental.pallas import tpu_sc as plsc

N = 10000
E = 320000
D = 128

NC = 2
NS = 16
NW = NC * NS
EPW = E // NW
K = 80
NCHUNK = EPW // K
NP = 10240
DSEG = NP // NS
RPS = NP // NS
ZROWS = 32

_MESH = plsc.VectorSubcoreMesh(core_axis_name="c", subcore_axis_name="s")


DEG_RING = 8
RCH = 25
RND = NCHUNK // RCH


@functools.partial(
    pl.kernel,
    out_type=jax.ShapeDtypeStruct((NC, NP), jnp.float32),
    mesh=_MESH,
    scratch_types=[
        pltpu.VMEM((NCHUNK, K), jnp.int32),
        pltpu.VMEM((K,), jnp.float32),
        pltpu.VMEM((DSEG,), jnp.float32),
        pltpu.VMEM_SHARED((NP,), jnp.float32),
        pltpu.SemaphoreType.DMA,
        pltpu.SemaphoreType.DMA,
    ],
)
def _deg_kernel(ei_hbm, out_hbm, didx2, ones_v, zbuf, dacc, isem, ssem):
    cid = lax.axis_index("c")
    sid = lax.axis_index("s")
    wid = sid * NC + cid
    zero16 = jnp.zeros((16,), jnp.float32)
    ones16 = jnp.ones((16,), jnp.float32)

    cidx = pltpu.async_copy(ei_hbm.at[1, wid], didx2, isem)

    def zfill(i, _):
        zbuf[pl.ds(i * 16, 16)] = zero16
        return 0

    lax.fori_loop(0, DSEG // 16, zfill, 0)

    def ofill(i, _):
        ones_v[pl.ds(i * 16, 16)] = ones16
        return 0

    lax.fori_loop(0, K // 16, ofill, 0)

    pltpu.sync_copy(zbuf, dacc.at[pl.ds(sid * DSEG, DSEG)])
    cidx.wait()
    plsc.subcore_barrier()

    def sc(j, _):
        pltpu.async_copy(ones_v, dacc.at[didx2.at[j]], ssem, add=True)

        @pl.when(j >= DEG_RING)
        def _():
            pltpu.make_async_copy(ones_v, dacc.at[didx2.at[0]], ssem).wait()

        return 0

    lax.fori_loop(0, NCHUNK, sc, 0)

    def drain(i, _):
        pltpu.make_async_copy(ones_v, dacc.at[didx2.at[0]], ssem).wait()
        return 0

    lax.fori_loop(0, DEG_RING, drain, 0)
    plsc.subcore_barrier()
    pltpu.sync_copy(dacc.at[pl.ds(sid * DSEG, DSEG)],
                    out_hbm.at[cid, pl.ds(sid * DSEG, DSEG)])


@functools.partial(
    pl.kernel,
    out_type=jax.ShapeDtypeStruct((NC, NP, D), jnp.float32),
    mesh=_MESH,
    scratch_types=[
        pltpu.VMEM((NCHUNK, K), jnp.int32),
        pltpu.VMEM((6, K), jnp.int32),
        pltpu.VMEM((K, D), jnp.float32),
        pltpu.VMEM((K, D), jnp.float32),
        pltpu.VMEM((K, D), jnp.float32),
        pltpu.VMEM_SHARED((NP, D), jnp.float32),
        pltpu.SemaphoreType.DMA,
        pltpu.SemaphoreType.DMA,
        pltpu.SemaphoreType.DMA,
        pltpu.SemaphoreType.DMA,
        pltpu.SemaphoreType.DMA,
        pltpu.SemaphoreType.DMA,
        pltpu.SemaphoreType.DMA,
        pltpu.SemaphoreType.DMA,
        pltpu.SemaphoreType.DMA,
        pltpu.SemaphoreType.DMA,
        pltpu.SemaphoreType.DMA,
        pltpu.SemaphoreType.DMA,
    ],
)
def _agg_kernel(h_hbm, ei_hbm, out_hbm,
                sidx, dring, buf0, buf1, buf2, acc,
                gsem0, gsem1, gsem2, ssem0, ssem1, ssem2,
                isem0, isem1, isem2, isem3, isem4, isem5):
    cid = lax.axis_index("c")
    sid = lax.axis_index("s")
    wid = sid * NC + cid
    zero16 = jnp.zeros((16,), jnp.float32)
    isems = (isem0, isem1, isem2, isem3, isem4, isem5)
    ssems = (ssem0, ssem1, ssem2)
    bufs = (buf0, buf1, buf2)

    cs = pltpu.async_copy(ei_hbm.at[0, wid], sidx, gsem0)
    for s in range(3):
        pltpu.async_copy(ei_hbm.at[1, wid, s], dring.at[s], isems[s])

    def zfill(i, _):
        buf0[i // (D // 16), pl.ds((i % (D // 16)) * 16, 16)] = zero16
        return 0

    lax.fori_loop(0, K * (D // 16), zfill, 0)

    def zcopy(j, _):
        pltpu.sync_copy(buf0, acc.at[pl.ds(sid * RPS + j * K, K)])
        return 0

    lax.fori_loop(0, RPS // K, zcopy, 0)
    cs.wait()
    plsc.subcore_barrier()

    def gather(j, buf, sem):
        jj = jnp.where(j < NCHUNK, j, 0)
        return pltpu.async_copy(h_hbm.at[sidx.at[jj]], buf, sem)

    def gwait(buf, sem):
        pltpu.make_async_copy(h_hbm.at[pl.ds(0, K)], buf, sem).wait()

    gather(0, buf0, gsem0)
    gather(1, buf1, gsem1)

    def step(j, slot):
        b = slot % 3
        nb = (slot + 2) % 3

        @pl.when(j + 2 <= NCHUNK)
        def _():
            @pl.when(j >= 1)
            def _():
                pltpu.make_async_copy(bufs[nb], acc.at[dring.at[nb]],
                                      ssems[nb]).wait()

            gather(j + 2, bufs[nb], (gsem0, gsem1, gsem2)[nb])

        gwait(bufs[b], (gsem0, gsem1, gsem2)[b])

        @pl.when(j < NCHUNK)
        def _():
            pltpu.make_async_copy(ei_hbm.at[1, wid, 0], dring.at[slot],
                                  isems[slot]).wait()
            pltpu.async_copy(bufs[b], acc.at[dring.at[slot]], ssems[b],
                             add=True)

        @pl.when(j + 3 < NCHUNK)
        def _():
            pltpu.async_copy(ei_hbm.at[1, wid, j + 3],
                             dring.at[(slot + 3) % 6], isems[(slot + 3) % 6])

    def hexa(t, _):
        j0 = 6 * t
        for s in range(6):
            step(j0 + s, s)
        return 0

    lax.fori_loop(0, (NCHUNK + 1) // 6, hexa, 0)
    pltpu.make_async_copy(buf0, acc.at[dring.at[0]], ssem0).wait()
    pltpu.make_async_copy(buf1, acc.at[dring.at[1]], ssem1).wait()

    plsc.subcore_barrier()
    pltpu.sync_copy(acc.at[pl.ds(sid * RPS, RPS)],
                    out_hbm.at[cid, pl.ds(sid * RPS, RPS)])


BN = 2000


def _scale_mm(x, w, degp):
    def body(x_ref, w_ref, dp_ref, h_ref, dinv_ref):
        deg = dp_ref[:, 0] + dp_ref[:, 1] + 1.0
        dinv = lax.rsqrt(deg)
        h = jnp.dot(x_ref[...], w_ref[...], preferred_element_type=jnp.float32)
        h_ref[...] = h * dinv[:, None]
        dinv_ref[...] = dinv[:, None]

    return pl.pallas_call(
        body,
        grid=(N // BN,),
        in_specs=[
            pl.BlockSpec((BN, D), lambda i: (i, 0)),
            pl.BlockSpec((D, D), lambda i: (0, 0)),
            pl.BlockSpec((BN, NC), lambda i: (i, 0)),
        ],
        out_specs=[
            pl.BlockSpec((BN, D), lambda i: (i, 0)),
            pl.BlockSpec((BN, 1), lambda i: (i, 0)),
        ],
        out_shape=[
            jax.ShapeDtypeStruct((N, D), jnp.float32),
            jax.ShapeDtypeStruct((N, 1), jnp.float32),
        ],
    )(x, w, degp)


def _combine_mm(acc, hp, dinv, b, w):
    def body(a_ref, hp_ref, dinv_ref, b_ref, w_ref, out_ref):
        dv = dinv_ref[...]
        u = dv * (a_ref[0] + a_ref[1] + hp_ref[...]) + b_ref[...]
        u = jnp.maximum(u, 0.0)
        out_ref[...] = jnp.dot(u, w_ref[...],
                               preferred_element_type=jnp.float32) * dv

    return pl.pallas_call(
        body,
        grid=(N // BN,),
        in_specs=[
            pl.BlockSpec((NC, BN, D), lambda i: (0, i, 0)),
            pl.BlockSpec((BN, D), lambda i: (i, 0)),
            pl.BlockSpec((BN, 1), lambda i: (i, 0)),
            pl.BlockSpec((1, D), lambda i: (0, 0)),
            pl.BlockSpec((D, D), lambda i: (0, 0)),
        ],
        out_specs=pl.BlockSpec((BN, D), lambda i: (i, 0)),
        out_shape=jax.ShapeDtypeStruct((N, D), jnp.float32),
    )(acc, hp, dinv, b, w)


def _final_mm(acc, hp, dinv, b, wo, bo):
    def body(a_ref, hp_ref, dinv_ref, b_ref, wo_ref, bo_ref, out_ref):
        dv = dinv_ref[...]
        v = dv * (a_ref[0] + a_ref[1] + hp_ref[...]) + b_ref[...]
        v = jnp.maximum(v, 0.0)
        out_ref[...] = jnp.dot(v, wo_ref[...],
                               preferred_element_type=jnp.float32) + bo_ref[...]

    return pl.pallas_call(
        body,
        grid=(N // BN,),
        in_specs=[
            pl.BlockSpec((NC, BN, D), lambda i: (0, i, 0)),
            pl.BlockSpec((BN, D), lambda i: (i, 0)),
            pl.BlockSpec((BN, 1), lambda i: (i, 0)),
            pl.BlockSpec((1, D), lambda i: (0, 0)),
            pl.BlockSpec((D, 1), lambda i: (0, 0)),
            pl.BlockSpec((1, 1), lambda i: (0, 0)),
        ],
        out_specs=pl.BlockSpec((BN, 1), lambda i: (i, 0)),
        out_shape=jax.ShapeDtypeStruct((N, 1), jnp.float32),
    )(acc, hp, dinv, b, wo, bo)


def kernel(x, edge_index, W1, b1, W2, b2, Wo, bo):
    ei = edge_index.astype(jnp.int32).reshape(2, NW, NCHUNK, K)

    degp = _deg_kernel(ei)
    h1, dinv = _scale_mm(x, W1, degp.T)
    acc1 = _agg_kernel(h1, ei)
    h2 = _combine_mm(acc1, h1, dinv, b1.reshape(1, D), W2)
    acc2 = _agg_kernel(h2, ei)
    out = _final_mm(acc2, h2, dinv, b2.reshape(1, D), Wo, bo.reshape(1, 1))
    return out.reshape(-1)


# speedup vs baseline: 39.5167x; 1.0228x over previous
import functools

import jax
import jax.numpy as jnp
from jax import lax
from jax.experimental import pallas as pl
from jax.experimental.pallas import tpu as pltpu
from jax.experimental.pallas import tpu_sc as plsc

N = 10000
E = 320000
D = 128

NC = 2
NS = 16
NW = NC * NS
EPW = E // NW
K = 80
NCHUNK = EPW // K
NP = 10240
DSEG = NP // NS
RPS = NP // NS
ZROWS = 32

_MESH = plsc.VectorSubcoreMesh(core_axis_name="c", subcore_axis_name="s")


DEG_RING = 8
RCH = 25
RND = NCHUNK // RCH


@functools.partial(
    pl.kernel,
    out_type=jax.ShapeDtypeStruct((NC, NP), jnp.float32),
    mesh=_MESH,
    scratch_types=[
        pltpu.VMEM((NCHUNK, K), jnp.int32),
        pltpu.VMEM((K,), jnp.float32),
        pltpu.VMEM((DSEG,), jnp.float32),
        pltpu.VMEM_SHARED((NP,), jnp.float32),
        pltpu.SemaphoreType.DMA,
        pltpu.SemaphoreType.DMA,
    ],
)
def _deg_kernel(ei_hbm, out_hbm, didx2, ones_v, zbuf, dacc, isem, ssem):
    cid = lax.axis_index("c")
    sid = lax.axis_index("s")
    wid = sid * NC + cid
    zero16 = jnp.zeros((16,), jnp.float32)
    ones16 = jnp.ones((16,), jnp.float32)

    cidx = pltpu.async_copy(ei_hbm.at[1, wid], didx2, isem)

    def zfill(i, _):
        zbuf[pl.ds(i * 16, 16)] = zero16
        return 0

    lax.fori_loop(0, DSEG // 16, zfill, 0)

    def ofill(i, _):
        ones_v[pl.ds(i * 16, 16)] = ones16
        return 0

    lax.fori_loop(0, K // 16, ofill, 0)

    pltpu.sync_copy(zbuf, dacc.at[pl.ds(sid * DSEG, DSEG)])
    cidx.wait()
    plsc.subcore_barrier()

    def sc(j, _):
        pltpu.async_copy(ones_v, dacc.at[didx2.at[j]], ssem, add=True)

        @pl.when(j >= DEG_RING)
        def _():
            pltpu.make_async_copy(ones_v, dacc.at[didx2.at[0]], ssem).wait()

        return 0

    lax.fori_loop(0, NCHUNK, sc, 0)

    def drain(i, _):
        pltpu.make_async_copy(ones_v, dacc.at[didx2.at[0]], ssem).wait()
        return 0

    lax.fori_loop(0, DEG_RING, drain, 0)
    plsc.subcore_barrier()
    pltpu.sync_copy(dacc.at[pl.ds(sid * DSEG, DSEG)],
                    out_hbm.at[cid, pl.ds(sid * DSEG, DSEG)])


@functools.partial(
    pl.kernel,
    out_type=jax.ShapeDtypeStruct((NC, NP, D), jnp.float32),
    mesh=_MESH,
    scratch_types=[
        pltpu.VMEM((NCHUNK, K), jnp.int32),
        pltpu.VMEM((6, K), jnp.int32),
        pltpu.VMEM((K, D), jnp.float32),
        pltpu.VMEM((K, D), jnp.float32),
        pltpu.VMEM((K, D), jnp.float32),
        pltpu.VMEM_SHARED((NP, D), jnp.float32),
        pltpu.SemaphoreType.DMA,
        pltpu.SemaphoreType.DMA,
        pltpu.SemaphoreType.DMA,
        pltpu.SemaphoreType.DMA,
        pltpu.SemaphoreType.DMA,
        pltpu.SemaphoreType.DMA,
        pltpu.SemaphoreType.DMA,
        pltpu.SemaphoreType.DMA,
        pltpu.SemaphoreType.DMA,
        pltpu.SemaphoreType.DMA,
        pltpu.SemaphoreType.DMA,
        pltpu.SemaphoreType.DMA,
    ],
)
def _agg_kernel(h_hbm, ei_hbm, out_hbm,
                sidx, dring, buf0, buf1, buf2, acc,
                gsem0, gsem1, gsem2, ssem0, ssem1, ssem2,
                isem0, isem1, isem2, isem3, isem4, isem5):
    cid = lax.axis_index("c")
    sid = lax.axis_index("s")
    wid = sid * NC + cid
    zero16 = jnp.zeros((16,), jnp.float32)
    isems = (isem0, isem1, isem2, isem3, isem4, isem5)
    ssems = (ssem0, ssem1, ssem2)
    bufs = (buf0, buf1, buf2)

    cs = pltpu.async_copy(ei_hbm.at[0, wid], sidx, gsem0)
    for s in range(3):
        pltpu.async_copy(ei_hbm.at[1, wid, s], dring.at[s], isems[s])

    def zfill(i, _):
        buf0[i // (D // 16), pl.ds((i % (D // 16)) * 16, 16)] = zero16
        return 0

    lax.fori_loop(0, K * (D // 16), zfill, 0)

    def zcopy(j, _):
        pltpu.async_copy(buf0, acc.at[pl.ds(sid * RPS + j * K, K)], ssem0)
        return 0

    lax.fori_loop(0, RPS // K, zcopy, 0)

    def zdrain(j, _):
        pltpu.make_async_copy(buf0, acc.at[pl.ds(sid * RPS, K)], ssem0).wait()
        return 0

    lax.fori_loop(0, RPS // K, zdrain, 0)
    cs.wait()
    plsc.subcore_barrier()

    def gather(j, buf, sem):
        jj = jnp.where(j < NCHUNK, j, 0)
        return pltpu.async_copy(h_hbm.at[sidx.at[jj]], buf, sem)

    def gwait(buf, sem):
        pltpu.make_async_copy(h_hbm.at[pl.ds(0, K)], buf, sem).wait()

    gather(0, buf0, gsem0)
    gather(1, buf1, gsem1)

    def step(j, slot):
        b = slot % 3
        nb = (slot + 2) % 3

        @pl.when(j + 2 <= NCHUNK)
        def _():
            @pl.when(j >= 1)
            def _():
                pltpu.make_async_copy(bufs[nb], acc.at[dring.at[nb]],
                                      ssems[nb]).wait()

            gather(j + 2, bufs[nb], (gsem0, gsem1, gsem2)[nb])

        gwait(bufs[b], (gsem0, gsem1, gsem2)[b])

        @pl.when(j < NCHUNK)
        def _():
            pltpu.make_async_copy(ei_hbm.at[1, wid, 0], dring.at[slot],
                                  isems[slot]).wait()
            pltpu.async_copy(bufs[b], acc.at[dring.at[slot]], ssems[b],
                             add=True)

        @pl.when(j + 3 < NCHUNK)
        def _():
            pltpu.async_copy(ei_hbm.at[1, wid, j + 3],
                             dring.at[(slot + 3) % 6], isems[(slot + 3) % 6])

    def hexa(t, _):
        j0 = 6 * t
        for s in range(6):
            step(j0 + s, s)
        return 0

    lax.fori_loop(0, (NCHUNK + 1) // 6, hexa, 0)
    pltpu.make_async_copy(buf0, acc.at[dring.at[0]], ssem0).wait()
    pltpu.make_async_copy(buf1, acc.at[dring.at[1]], ssem1).wait()

    plsc.subcore_barrier()
    pltpu.sync_copy(acc.at[pl.ds(sid * RPS, RPS)],
                    out_hbm.at[cid, pl.ds(sid * RPS, RPS)])


BN = 2000


def _scale_mm(x, w, degp):
    def body(x_ref, w_ref, dp_ref, h_ref, dinv_ref):
        deg = dp_ref[:, 0] + dp_ref[:, 1] + 1.0
        dinv = lax.rsqrt(deg)
        h = jnp.dot(x_ref[...], w_ref[...], preferred_element_type=jnp.float32)
        h_ref[...] = h * dinv[:, None]
        dinv_ref[...] = dinv[:, None]

    return pl.pallas_call(
        body,
        grid=(N // BN,),
        in_specs=[
            pl.BlockSpec((BN, D), lambda i: (i, 0)),
            pl.BlockSpec((D, D), lambda i: (0, 0)),
            pl.BlockSpec((BN, NC), lambda i: (i, 0)),
        ],
        out_specs=[
            pl.BlockSpec((BN, D), lambda i: (i, 0)),
            pl.BlockSpec((BN, 1), lambda i: (i, 0)),
        ],
        out_shape=[
            jax.ShapeDtypeStruct((N, D), jnp.float32),
            jax.ShapeDtypeStruct((N, 1), jnp.float32),
        ],
    )(x, w, degp)


def _combine_mm(acc, hp, dinv, b, w):
    def body(a_ref, hp_ref, dinv_ref, b_ref, w_ref, out_ref):
        dv = dinv_ref[...]
        u = dv * (a_ref[0] + a_ref[1] + hp_ref[...]) + b_ref[...]
        u = jnp.maximum(u, 0.0)
        out_ref[...] = jnp.dot(u, w_ref[...],
                               preferred_element_type=jnp.float32) * dv

    return pl.pallas_call(
        body,
        grid=(N // BN,),
        in_specs=[
            pl.BlockSpec((NC, BN, D), lambda i: (0, i, 0)),
            pl.BlockSpec((BN, D), lambda i: (i, 0)),
            pl.BlockSpec((BN, 1), lambda i: (i, 0)),
            pl.BlockSpec((1, D), lambda i: (0, 0)),
            pl.BlockSpec((D, D), lambda i: (0, 0)),
        ],
        out_specs=pl.BlockSpec((BN, D), lambda i: (i, 0)),
        out_shape=jax.ShapeDtypeStruct((N, D), jnp.float32),
    )(acc, hp, dinv, b, w)


def _final_mm(acc, hp, dinv, b, wo, bo):
    def body(a_ref, hp_ref, dinv_ref, b_ref, wo_ref, bo_ref, out_ref):
        dv = dinv_ref[...]
        v = dv * (a_ref[0] + a_ref[1] + hp_ref[...]) + b_ref[...]
        v = jnp.maximum(v, 0.0)
        out_ref[...] = lax.dot_general(
            wo_ref[...], v, (((0,), (1,)), ((), ())),
            preferred_element_type=jnp.float32) + bo_ref[...]

    return pl.pallas_call(
        body,
        grid=(1,),
        in_specs=[
            pl.BlockSpec((NC, N, D), lambda i: (0, 0, 0)),
            pl.BlockSpec((N, D), lambda i: (0, 0)),
            pl.BlockSpec((N, 1), lambda i: (0, 0)),
            pl.BlockSpec((1, D), lambda i: (0, 0)),
            pl.BlockSpec((D, 1), lambda i: (0, 0)),
            pl.BlockSpec((1, 1), lambda i: (0, 0)),
        ],
        out_specs=pl.BlockSpec((1, N), lambda i: (0, 0)),
        out_shape=jax.ShapeDtypeStruct((1, N), jnp.float32),
    )(acc, hp, dinv, b, wo, bo)


def kernel(x, edge_index, W1, b1, W2, b2, Wo, bo):
    ei = edge_index.astype(jnp.int32).reshape(2, NW, NCHUNK, K)

    degp = _deg_kernel(ei)
    h1, dinv = _scale_mm(x, W1, degp.T)
    acc1 = _agg_kernel(h1, ei)
    h2 = _combine_mm(acc1, h1, dinv, b1.reshape(1, D), W2)
    acc2 = _agg_kernel(h2, ei)
    out = _final_mm(acc2, h2, dinv, b2.reshape(1, D), Wo, bo.reshape(1, 1))
    return out.reshape(-1)
